# baseline (speedup 1.0000x reference)
import functools

import jax
import jax.numpy as jnp
from jax import lax
from jax.experimental import pallas as pl
from jax.experimental.pallas import tpu as pltpu
from jax.experimental.pallas import tpu_sc as plsc

MARGIN_ = 1.0
EPS_ = 1e-6
CB_ = 1024
KAUG_ = 128
NC_, NS_ = 2, 16
NW_ = NC_ * NS_


def _sc_gather_body(table_hbm, idx_hbm, out_hbm, idx_v, rows_v, sem, *, bpw):
    wid = lax.axis_index("s") * NC_ + lax.axis_index("c")
    base = wid * bpw
    pltpu.sync_copy(idx_hbm.at[pl.ds(base, bpw)], idx_v)
    pltpu.async_copy(table_hbm.at[idx_v], rows_v, sem).wait()
    pltpu.sync_copy(rows_v, out_hbm.at[pl.ds(base, bpw)])


def _gather_positives(target, labels):
    B = labels.shape[0]
    D = target.shape[1]
    bpw = B // NW_
    mesh = plsc.VectorSubcoreMesh(core_axis_name="c", subcore_axis_name="s")
    return pl.kernel(
        functools.partial(_sc_gather_body, bpw=bpw),
        mesh=mesh,
        out_type=jax.ShapeDtypeStruct((B, D), jnp.float32),
        scratch_types=[
            pltpu.VMEM((bpw,), jnp.int32),
            pltpu.VMEM((bpw, D), jnp.float32),
            pltpu.SemaphoreType.DMA,
        ],
        compiler_params=pltpu.CompilerParams(use_tc_tiling_on_sc=False),
    )(target, labels)


def _dist_body(a_aug_ref, labels_ref, target_hbm, minacc_ref, t_buf, sem,
               *, n_valid, nblocks):
    i = pl.program_id(0)
    B = a_aug_ref.shape[0]
    D = t_buf.shape[3]
    G = CB_ // 8
    slot = lax.rem(i, 2)

    def _start(idx):
        return jnp.where(idx == nblocks - 1, (n_valid - CB_) // 8, idx * G)

    @pl.when(i == 0)
    def _prime():
        pltpu.make_async_copy(
            target_hbm.at[pl.ds(0, G)], t_buf.at[0], sem.at[0]).start()

    @pl.when(i + 1 < nblocks)
    def _prefetch():
        pltpu.make_async_copy(
            target_hbm.at[pl.ds(_start(i + 1), G)],
            t_buf.at[1 - slot], sem.at[1 - slot]).start()

    pltpu.make_async_copy(
        target_hbm.at[pl.ds(_start(i), G)], t_buf.at[slot],
        sem.at[slot]).wait()

    t = t_buf[slot].reshape(CB_, D) - EPS_
    t_sq = jnp.sum(t * t, axis=1, keepdims=True)
    t_aug = jnp.concatenate(
        [t * -2.0, t_sq, jnp.zeros((CB_, KAUG_ - D - 1), jnp.float32)],
        axis=1)

    s = lax.dot_general(a_aug_ref[...], t_aug, (((1,), (1,)), ((), ())),
                        preferred_element_type=jnp.float32)

    @pl.when(i == 0)
    def _init():
        minacc_ref[...] = jnp.full_like(minacc_ref, jnp.inf)

    lbl_s = labels_ref[...] - _start(i) * 8
    lane = lax.broadcasted_iota(jnp.int32, (B, 128), 1)
    m = minacc_ref[...]
    for k in range(CB_ // 128):
        sk = s[:, k * 128:(k + 1) * 128]
        own = (lane + k * 128) == lbl_s
        m = jnp.minimum(m, jnp.where(own, jnp.inf, sk))
    minacc_ref[...] = m


def _final_body(minacc_ref, inputs_ref, pos_ref, out_ref):
    a = inputs_ref[...]
    a_sq = jnp.sum(a * a, axis=1, keepdims=True)
    d_an = jnp.sqrt(jnp.clip(
        a_sq + jnp.min(minacc_ref[...], axis=1, keepdims=True), 1e-12))
    dp = a - pos_ref[...] + EPS_
    d_ap = jnp.sqrt(jnp.clip(jnp.sum(dp * dp, axis=1, keepdims=True), 1e-12))
    per = jnp.maximum(d_ap - d_an + MARGIN_, 0.0)
    out_ref[0, 0] = jnp.sum(per) / a.shape[0]


def kernel(inputs, labels, target):
    B, D = inputs.shape
    C = target.shape[0]
    nblocks = (C + CB_ - 1) // CB_

    pos = _gather_positives(target, labels)

    a_aug = jnp.concatenate(
        [inputs,
         jnp.ones((B, 1), jnp.float32),
         jnp.zeros((B, KAUG_ - D - 1), jnp.float32)], axis=1)
    labels2 = labels.reshape(B, 1)

    minacc = pl.pallas_call(
        functools.partial(_dist_body, n_valid=C, nblocks=nblocks),
        grid=(nblocks,),
        in_specs=[
            pl.BlockSpec((B, KAUG_), lambda i: (0, 0)),
            pl.BlockSpec((B, 1), lambda i: (0, 0)),
            pl.BlockSpec(memory_space=pl.ANY),
        ],
        out_specs=pl.BlockSpec((B, 128), lambda i: (0, 0)),
        out_shape=jax.ShapeDtypeStruct((B, 128), jnp.float32),
        scratch_shapes=[
            pltpu.VMEM((2, CB_ // 8, 8, D), jnp.float32),
            pltpu.SemaphoreType.DMA((2,)),
        ],
        compiler_params=pltpu.CompilerParams(
            dimension_semantics=("arbitrary",)),
    )(a_aug, labels2, target.reshape(C // 8, 8, D))

    out = pl.pallas_call(
        _final_body,
        out_specs=pl.BlockSpec(memory_space=pltpu.SMEM),
        out_shape=jax.ShapeDtypeStruct((1, 1), jnp.float32),
    )(minacc, inputs, pos)
    return out[0, 0]
